# baseline (speedup 1.0000x reference)
import jax
import jax.numpy as jnp
from jax import lax
from jax.experimental import pallas as pl
from jax.experimental.pallas import tpu as pltpu
from jax.experimental.pallas import tpu_sc as plsc

B = 16384
H = 50
E = 32
NUM_CORES = 2
NUM_SUBCORES = 16
NW = NUM_CORES * NUM_SUBCORES
B_PER_W = B // NW
CH = 256
DT = E // 8
CBTL = CH // 128


def _emb_body(ids_hbm, table_hbm, out_hbm, idx_v, idx2_a, idx2_b,
              rows_a, rows_b, trans_a, trans_b,
              sem_a, sem_b, sem_wa, sem_wb):
    wid = lax.axis_index("s") * NUM_CORES + lax.axis_index("c")
    b0 = wid * B_PER_W
    pltpu.sync_copy(ids_hbm.at[:, pl.ds(b0, B_PER_W)], idx_v)
    iota = lax.iota(jnp.int32, 16)
    s_v = iota % 8
    dt0_v = iota // 8

    def shift_idx(h, c, idx2):
        def sh(i, _):
            idx2[pl.ds(i * 16, 16)] = \
                jax.lax.shift_right_logical(idx_v[h, pl.ds(c * CH + i * 16, 16)], 2)
            return 0
        lax.fori_loop(0, CH // 16, sh, 0)

    def transpose(h, c, rows, trans):
        for btl in range(CBTL):
            btl_v = jnp.full((16,), btl, jnp.int32)

            @plsc.parallel_loop(0, 128 // 16, unroll=2)
            def tr_l(li0, btl=btl, btl_v=btl_v):
                ivec = idx_v[h, pl.ds(c * CH + btl * 128 + li0 * 16, 16)]
                qvec = (ivec & 3) * E
                for dl in range(16):
                    li = li0 * 16 + dl
                    l_v = jnp.full((16,), li, jnp.int32)
                    row = btl * 128 + li
                    qoff = qvec[dl]
                    for k in range(2):
                        v = rows[row, pl.ds(qoff + k * 16, 16)]
                        plsc.store_scatter(
                            trans, [dt0_v + 2 * k, btl_v, s_v, l_v], v)

    def out_copy(trans, h, c, sem):
        return pltpu.make_async_copy(
            trans.at[:, :, pl.ds(0, 8), pl.ds(0, 128)],
            out_hbm.at[h, :, pl.ds(b0 // 128 + c * CBTL, CBTL)], sem)

    def gather(h, c, idx2, rows, sem):
        shift_idx(h, c, idx2)
        return pltpu.async_copy(table_hbm.at[idx2], rows, sem)

    gather(0, 0, idx2_a, rows_a, sem_a)
    gather(0, 1, idx2_b, rows_b, sem_b)

    def step(h, _):
        pltpu.make_async_copy(table_hbm.at[idx2_a], rows_a, sem_a).wait()

        @pl.when(h > 0)
        def _():
            out_copy(trans_a, h - 1, 0, sem_wa).wait()

        transpose(h, 0, rows_a, trans_a)
        out_copy(trans_a, h, 0, sem_wa).start()

        @pl.when(h + 1 < H)
        def _():
            gather(h + 1, 0, idx2_a, rows_a, sem_a)

        pltpu.make_async_copy(table_hbm.at[idx2_b], rows_b, sem_b).wait()

        @pl.when(h > 0)
        def _():
            out_copy(trans_b, h - 1, 1, sem_wb).wait()

        transpose(h, 1, rows_b, trans_b)
        out_copy(trans_b, h, 1, sem_wb).start()

        @pl.when(h + 1 < H)
        def _():
            gather(h + 1, 1, idx2_b, rows_b, sem_b)

        return 0

    lax.fori_loop(0, H, step, 0)
    out_copy(trans_a, H - 1, 0, sem_wa).wait()
    out_copy(trans_b, H - 1, 1, sem_wb).wait()


def kernel(input_ids, weight):
    ids_t = input_ids.T
    table4 = weight.reshape(weight.shape[0] * E // 128, 128)
    mesh = plsc.VectorSubcoreMesh(core_axis_name="c", subcore_axis_name="s")
    out5 = pl.kernel(
        _emb_body,
        out_type=jax.ShapeDtypeStruct((H, DT, B // 128, 8, 128), jnp.float32),
        mesh=mesh,
        scratch_types=[
            pltpu.VMEM((H, B_PER_W), jnp.int32),
            pltpu.VMEM((CH,), jnp.int32),
            pltpu.VMEM((CH,), jnp.int32),
            pltpu.VMEM((CH, 128), jnp.float32),
            pltpu.VMEM((CH, 128), jnp.float32),
            pltpu.VMEM((DT, CBTL, 10, 129), jnp.float32),
            pltpu.VMEM((DT, CBTL, 10, 129), jnp.float32),
            pltpu.SemaphoreType.DMA,
            pltpu.SemaphoreType.DMA,
            pltpu.SemaphoreType.DMA,
            pltpu.SemaphoreType.DMA,
        ],
        compiler_params=pltpu.CompilerParams(use_tc_tiling_on_sc=False,
                                             needs_layout_passes=False),
    )(ids_t, table4)
    return out5.transpose(2, 4, 0, 1, 3).reshape(B, H, E)


# speedup vs baseline: 1.0079x; 1.0079x over previous
import jax
import jax.numpy as jnp
from jax import lax
from jax.experimental import pallas as pl
from jax.experimental.pallas import tpu as pltpu
from jax.experimental.pallas import tpu_sc as plsc

B = 16384
H = 50
E = 32
NUM_CORES = 2
NUM_SUBCORES = 16
NW = NUM_CORES * NUM_SUBCORES
B_PER_W = B // NW
CH = 256
DT = E // 8
CBTL = CH // 128


def _emb_body(ids_hbm, table_hbm, out_hbm, idx_v, idx2_a, idx2_b,
              rows_a, rows_b, trans_a, trans_b,
              sem_a, sem_b, sem_wa, sem_wb):
    wid = lax.axis_index("s") * NUM_CORES + lax.axis_index("c")
    b0 = wid * B_PER_W
    pltpu.sync_copy(ids_hbm.at[:, pl.ds(b0, B_PER_W)], idx_v)
    iota = lax.iota(jnp.int32, 16)
    s_v = iota % 8
    dt0_v = iota // 8

    def shift_idx(h, c, idx2):
        def sh(i, _):
            idx2[pl.ds(i * 16, 16)] = \
                jax.lax.shift_right_logical(idx_v[h, pl.ds(c * CH + i * 16, 16)], 2)
            return 0
        lax.fori_loop(0, CH // 16, sh, 0)

    def transpose(h, c, rows, trans):
        for btl in range(CBTL):
            btl_v = jnp.full((16,), btl, jnp.int32)

            def tr_l(li0, _, btl=btl, btl_v=btl_v):
                ivec = idx_v[h, pl.ds(c * CH + btl * 128 + li0 * 16, 16)]
                qvec = (ivec & 3) * E
                for dl in range(16):
                    li = li0 * 16 + dl
                    l_v = jnp.full((16,), li, jnp.int32)
                    row = btl * 128 + li
                    qoff = qvec[dl]
                    for k in range(2):
                        v = rows[row, pl.ds(qoff + k * 16, 16)]
                        plsc.store_scatter(
                            trans, [dt0_v + 2 * k, btl_v, s_v, l_v], v)
                return 0
            lax.fori_loop(0, 128 // 16, tr_l, 0)

    def out_copy(trans, h, c, sem):
        return pltpu.make_async_copy(
            trans.at[:, :, pl.ds(0, 8), pl.ds(0, 128)],
            out_hbm.at[h, :, pl.ds(b0 // 128 + c * CBTL, CBTL)], sem)

    def gather(h, c, idx2, rows, sem):
        shift_idx(h, c, idx2)
        return pltpu.async_copy(table_hbm.at[idx2], rows, sem)

    gather(0, 0, idx2_a, rows_a, sem_a)
    gather(0, 1, idx2_b, rows_b, sem_b)

    def step(h, _):
        pltpu.make_async_copy(table_hbm.at[idx2_a], rows_a, sem_a).wait()

        @pl.when(h > 0)
        def _():
            out_copy(trans_a, h - 1, 0, sem_wa).wait()

        transpose(h, 0, rows_a, trans_a)
        out_copy(trans_a, h, 0, sem_wa).start()

        @pl.when(h + 1 < H)
        def _():
            gather(h + 1, 0, idx2_a, rows_a, sem_a)

        pltpu.make_async_copy(table_hbm.at[idx2_b], rows_b, sem_b).wait()

        @pl.when(h > 0)
        def _():
            out_copy(trans_b, h - 1, 1, sem_wb).wait()

        transpose(h, 1, rows_b, trans_b)
        out_copy(trans_b, h, 1, sem_wb).start()

        @pl.when(h + 1 < H)
        def _():
            gather(h + 1, 1, idx2_b, rows_b, sem_b)

        return 0

    lax.fori_loop(0, H, step, 0)
    out_copy(trans_a, H - 1, 0, sem_wa).wait()
    out_copy(trans_b, H - 1, 1, sem_wb).wait()


def kernel(input_ids, weight):
    ids_t = input_ids.T
    table4 = weight.reshape(weight.shape[0] * E // 128, 128)
    mesh = plsc.VectorSubcoreMesh(core_axis_name="c", subcore_axis_name="s")
    out5 = pl.kernel(
        _emb_body,
        out_type=jax.ShapeDtypeStruct((H, DT, B // 128, 8, 128), jnp.float32),
        mesh=mesh,
        scratch_types=[
            pltpu.VMEM((H, B_PER_W), jnp.int32),
            pltpu.VMEM((CH,), jnp.int32),
            pltpu.VMEM((CH,), jnp.int32),
            pltpu.VMEM((CH, 128), jnp.float32),
            pltpu.VMEM((CH, 128), jnp.float32),
            pltpu.VMEM((DT, CBTL, 10, 129), jnp.float32),
            pltpu.VMEM((DT, CBTL, 10, 129), jnp.float32),
            pltpu.SemaphoreType.DMA,
            pltpu.SemaphoreType.DMA,
            pltpu.SemaphoreType.DMA,
            pltpu.SemaphoreType.DMA,
        ],
        compiler_params=pltpu.CompilerParams(use_tc_tiling_on_sc=False,
                                             needs_layout_passes=False),
    )(ids_t, table4)
    return out5.transpose(2, 4, 0, 1, 3).reshape(B, H, E)
